# Initial kernel scaffold; baseline (speedup 1.0000x reference)
import jax
import jax.numpy as jnp
from jax.experimental import pallas as pl


def kernel(x, weight):
    raise NotImplementedError("write your pallas kernel here")


# speedup vs baseline: 1.8293x; 1.8293x over previous
import functools

import jax
import jax.numpy as jnp
from jax import lax
from jax.experimental import pallas as pl
from jax.experimental.pallas import tpu as pltpu
from jax.experimental.pallas import tpu_sc as plsc

_VOCAB = 1000000
_N_HIDDEN = 64
_BATCH = 16384
_HIST = 50
_TOTAL = _BATCH * _HIST

_NC = 2
_NS = 16
_NW = _NC * _NS
_B_PER_W = _TOTAL // _NW
_CHUNK = 800
_N_CHUNKS = _B_PER_W // _CHUNK

_mesh = plsc.VectorSubcoreMesh(core_axis_name="c", subcore_axis_name="s")


@functools.partial(
    pl.kernel,
    mesh=_mesh,
    out_type=jax.ShapeDtypeStruct((_TOTAL, _N_HIDDEN), jnp.float32),
    scratch_types=[
        pltpu.VMEM((_CHUNK,), jnp.int32),
        pltpu.VMEM((_CHUNK, _N_HIDDEN), jnp.float32),
        pltpu.SemaphoreType.DMA,
    ],
    compiler_params=pltpu.CompilerParams(use_tc_tiling_on_sc=False),
)
def _gather_kernel(idx_hbm, table_hbm, out_hbm, idx_v, rows_v, sem):
    wid = lax.axis_index("s") * _NC + lax.axis_index("c")
    base = wid * _B_PER_W

    def body(i, carry):
        off = base + i * _CHUNK
        pltpu.sync_copy(idx_hbm.at[pl.ds(off, _CHUNK)], idx_v)
        pltpu.async_copy(table_hbm.at[idx_v], rows_v, sem).wait()
        pltpu.sync_copy(rows_v, out_hbm.at[pl.ds(off, _CHUNK)])
        return carry

    lax.fori_loop(0, _N_CHUNKS, body, 0)


def kernel(x, weight):
    flat = x.reshape(_TOTAL).astype(jnp.int32)
    out = _gather_kernel(flat, weight)
    return out.reshape(_BATCH, _HIST, _N_HIDDEN)


# speedup vs baseline: 1.8709x; 1.0227x over previous
import functools

import jax
import jax.numpy as jnp
from jax import lax
from jax.experimental import pallas as pl
from jax.experimental.pallas import tpu as pltpu
from jax.experimental.pallas import tpu_sc as plsc

_VOCAB = 1000000
_N_HIDDEN = 64
_BATCH = 16384
_HIST = 50
_TOTAL = _BATCH * _HIST

_NC = 2
_NS = 16
_NW = _NC * _NS
_B_PER_W = _TOTAL // _NW
_CHUNK = 800
_N_CHUNKS = _B_PER_W // _CHUNK
_N_PAIRS = _N_CHUNKS // 2

_mesh = plsc.VectorSubcoreMesh(core_axis_name="c", subcore_axis_name="s")


@functools.partial(
    pl.kernel,
    mesh=_mesh,
    out_type=jax.ShapeDtypeStruct((_TOTAL, _N_HIDDEN), jnp.float32),
    scratch_types=[
        pltpu.VMEM((_B_PER_W,), jnp.int32),
        pltpu.VMEM((_CHUNK, _N_HIDDEN), jnp.float32),
        pltpu.VMEM((_CHUNK, _N_HIDDEN), jnp.float32),
        pltpu.SemaphoreType.DMA,
        pltpu.SemaphoreType.DMA,
        pltpu.SemaphoreType.DMA,
        pltpu.SemaphoreType.DMA,
    ],
    compiler_params=pltpu.CompilerParams(use_tc_tiling_on_sc=False),
)
def _gather_kernel(idx_hbm, table_hbm, out_hbm, idx_all, rows0, rows1,
                   sg0, sg1, so0, so1):
    wid = lax.axis_index("s") * _NC + lax.axis_index("c")
    base = wid * _B_PER_W

    pltpu.sync_copy(idx_hbm.at[pl.ds(base, _B_PER_W)], idx_all)

    def start_gather(i, rows, sem):
        pltpu.async_copy(
            table_hbm.at[idx_all.at[pl.ds(i * _CHUNK, _CHUNK)]], rows, sem)

    def wait_gather(i, rows, sem):
        pltpu.make_async_copy(
            table_hbm.at[idx_all.at[pl.ds(i * _CHUNK, _CHUNK)]], rows,
            sem).wait()

    def start_store(i, rows, sem):
        pltpu.async_copy(rows, out_hbm.at[pl.ds(base + i * _CHUNK, _CHUNK)],
                         sem)

    def wait_store(i, rows, sem):
        pltpu.make_async_copy(
            rows, out_hbm.at[pl.ds(base + i * _CHUNK, _CHUNK)], sem).wait()

    start_gather(0, rows0, sg0)

    def body(j, carry):
        c0 = 2 * j
        c1 = 2 * j + 1

        @pl.when(j > 0)
        def _():
            wait_store(c0 - 1, rows1, so1)

        start_gather(c1, rows1, sg1)
        wait_gather(c0, rows0, sg0)
        start_store(c0, rows0, so0)

        @pl.when(j < _N_PAIRS - 1)
        def _():
            wait_store(c0, rows0, so0)
            start_gather(c0 + 2, rows0, sg0)

        wait_gather(c1, rows1, sg1)
        start_store(c1, rows1, so1)
        return carry

    lax.fori_loop(0, _N_PAIRS, body, 0)

    wait_store(_N_CHUNKS - 2, rows0, so0)
    wait_store(_N_CHUNKS - 1, rows1, so1)


def kernel(x, weight):
    flat = x.reshape(_TOTAL).astype(jnp.int32)
    out = _gather_kernel(flat, weight)
    return out.reshape(_BATCH, _HIST, _N_HIDDEN)


# speedup vs baseline: 1.8760x; 1.0027x over previous
import functools

import jax
import jax.numpy as jnp
from jax import lax
from jax.experimental import pallas as pl
from jax.experimental.pallas import tpu as pltpu
from jax.experimental.pallas import tpu_sc as plsc

_VOCAB = 1000000
_N_HIDDEN = 64
_BATCH = 16384
_HIST = 50
_TOTAL = _BATCH * _HIST

_NC = 2
_NS = 16
_NW = _NC * _NS
_B_PER_W = _TOTAL // _NW
_CHUNK = 400
_N_CHUNKS = _B_PER_W // _CHUNK
_NBUF = 4
_N_GROUPS = _N_CHUNKS // _NBUF

_mesh = plsc.VectorSubcoreMesh(core_axis_name="c", subcore_axis_name="s")


@functools.partial(
    pl.kernel,
    mesh=_mesh,
    out_type=jax.ShapeDtypeStruct((_TOTAL, _N_HIDDEN), jnp.float32),
    scratch_types=[
        pltpu.VMEM((_B_PER_W,), jnp.int32),
        [pltpu.VMEM((_CHUNK, _N_HIDDEN), jnp.float32)] * _NBUF,
        [pltpu.SemaphoreType.DMA] * _NBUF,
        [pltpu.SemaphoreType.DMA] * _NBUF,
    ],
    compiler_params=pltpu.CompilerParams(use_tc_tiling_on_sc=False),
)
def _gather_kernel(idx_hbm, table_hbm, out_hbm, idx_all, bufs, sgs, sos):
    wid = lax.axis_index("s") * _NC + lax.axis_index("c")
    base = wid * _B_PER_W

    pltpu.sync_copy(idx_hbm.at[pl.ds(base, _B_PER_W)], idx_all)

    def start_gather(i, b):
        pltpu.async_copy(
            table_hbm.at[idx_all.at[pl.ds(i * _CHUNK, _CHUNK)]], bufs[b],
            sgs[b])

    def wait_gather(i, b):
        pltpu.make_async_copy(
            table_hbm.at[idx_all.at[pl.ds(i * _CHUNK, _CHUNK)]], bufs[b],
            sgs[b]).wait()

    def start_store(i, b):
        pltpu.async_copy(bufs[b], out_hbm.at[pl.ds(base + i * _CHUNK, _CHUNK)],
                         sos[b])

    def wait_store(i, b):
        pltpu.make_async_copy(
            bufs[b], out_hbm.at[pl.ds(base + i * _CHUNK, _CHUNK)],
            sos[b]).wait()

    for b in range(_NBUF):
        start_gather(b, b)

    def body(j, carry):
        for b in range(_NBUF):
            i = _NBUF * j + b
            wait_gather(i, b)
            start_store(i, b)
            ip = i - 1
            pb = (b - 1) % _NBUF

            @pl.when((ip >= 0) & (ip < _N_CHUNKS - _NBUF))
            def _():
                wait_store(ip, pb)
                start_gather(ip + _NBUF, pb)

        return carry

    lax.fori_loop(0, _N_GROUPS, body, 0)

    for b in range(_NBUF):
        wait_store(_N_CHUNKS - _NBUF + b, b)


def kernel(x, weight):
    flat = x.reshape(_TOTAL).astype(jnp.int32)
    out = _gather_kernel(flat, weight)
    return out.reshape(_BATCH, _HIST, _N_HIDDEN)
